# baseline (device time: 123042 ns/iter reference)
import jax
import jax.numpy as jnp
from jax import lax
from jax.experimental import pallas as pl
from jax.experimental.pallas import tpu as pltpu

T = 4096
V_SHARD = 8192
D = 2048
C = T // 4
S = 8
SB = C // S

_MESH = pl.DeviceIdType.MESH


def _fused_kernel(ids, ids_col, E):
    def body(ids_sref, idv_ref, E_ref, out_ref,
             gbuf, pbuf, comp_ref, ry_ref,
             gsems, osem, ysend, yrecv, xsend, xrecv, zsend, zrecv,
             dsend, drecv):
        my_x = lax.axis_index("x")
        my_y = lax.axis_index("y")
        my_z = lax.axis_index("z")
        ypeer = (my_x, 1 - my_y, my_z)
        xpeer = (1 - my_x, my_y, my_z)
        zpeer = (my_x, my_y, 1 - my_z)

        base = (2 * my_x + my_z) * C
        off = my_y * V_SHARD
        row_me = (2 * my_x + my_z) * C
        row_x = (2 * (1 - my_x) + my_z) * C
        row_z = (2 * my_x + (1 - my_z)) * C

        barrier_sem = pltpu.get_barrier_semaphore()
        for nbr in (ypeer, xpeer, zpeer):
            pl.semaphore_signal(barrier_sem, inc=1, device_id=nbr,
                                device_id_type=_MESH)
        pl.semaphore_wait(barrier_sem, 3)

        def issue_sub(s):
            def f(j, cnt):
                idx = ids_sref[base + s * SB + j] - off
                owned = (idx >= 0) & (idx < V_SHARD)

                @pl.when(owned)
                def _():
                    pltpu.make_async_copy(
                        E_ref.at[pl.ds(idx, 1), :],
                        gbuf.at[pl.ds(s * SB + j, 1), :],
                        gsems.at[s],
                    ).start()

                return cnt + owned.astype(jnp.int32)

            return lax.fori_loop(0, SB, f, 0, unroll=8)

        def wait_gather(s, count):
            def w(_, __):
                pltpu.make_async_copy(
                    E_ref.at[pl.ds(0, 1), :],
                    gbuf.at[pl.ds(0, 1), :],
                    gsems.at[s],
                ).wait()
                return 0

            lax.fori_loop(0, count, w, 0)

        counts = [issue_sub(0)]
        y_rdmas = []
        for s in range(S):
            sl = pl.ds(s * SB, SB)
            wait_gather(s, counts[s])
            idv = idv_ref[pl.ds(base + s * SB, SB), :]
            owned = (idv >= off) & (idv < off + V_SHARD)
            pbuf[sl, :] = jnp.where(
                owned, gbuf[sl, :], 0.0
            ).astype(jnp.bfloat16)
            r = pltpu.make_async_remote_copy(
                src_ref=pbuf.at[sl], dst_ref=ry_ref.at[sl],
                send_sem=ysend.at[s], recv_sem=yrecv.at[s],
                device_id=ypeer, device_id_type=_MESH,
            )
            r.start()
            y_rdmas.append(r)
            if s + 1 < S:
                counts.append(issue_sub(s + 1))

        n_ostores = 0
        g_rdmas = []
        for s in range(S):
            sl = pl.ds(s * SB, SB)
            osl = pl.ds(row_me + s * SB, SB)
            y_rdmas[s].wait_recv()
            comp_ref[sl, :] = pbuf[sl, :] + ry_ref[sl, :]
            gx = pltpu.make_async_remote_copy(
                src_ref=comp_ref.at[sl], dst_ref=out_ref.at[osl],
                send_sem=xsend.at[s], recv_sem=xrecv.at[s],
                device_id=xpeer, device_id_type=_MESH,
            )
            gz = pltpu.make_async_remote_copy(
                src_ref=comp_ref.at[sl], dst_ref=out_ref.at[osl],
                send_sem=zsend.at[s], recv_sem=zrecv.at[s],
                device_id=zpeer, device_id_type=_MESH,
            )
            gx.start()
            gz.start()
            pltpu.make_async_copy(
                comp_ref.at[sl], out_ref.at[osl], osem
            ).start()
            n_ostores += 1
            g_rdmas.append((gx, gz))

        d_rdmas = []
        for s in range(S):
            gx, gz = g_rdmas[s]
            xsl = pl.ds(row_x + s * SB, SB)
            zsl = pl.ds(row_z + s * SB, SB)
            gx.wait_recv()
            if s % 2 == 0:
                d = pltpu.make_async_remote_copy(
                    src_ref=out_ref.at[xsl], dst_ref=out_ref.at[xsl],
                    send_sem=dsend.at[s], recv_sem=drecv.at[s],
                    device_id=zpeer, device_id_type=_MESH,
                )
                d.start()
            gz.wait_recv()
            if s % 2 == 1:
                d = pltpu.make_async_remote_copy(
                    src_ref=out_ref.at[zsl], dst_ref=out_ref.at[zsl],
                    send_sem=dsend.at[s], recv_sem=drecv.at[s],
                    device_id=xpeer, device_id_type=_MESH,
                )
                d.start()
            d_rdmas.append(d)

        for d in d_rdmas:
            d.wait_recv()
        for _ in range(n_ostores):
            pltpu.make_async_copy(
                comp_ref.at[pl.ds(0, SB)],
                out_ref.at[pl.ds(0, SB), :],
                osem,
            ).wait()
        for r in y_rdmas:
            r.wait_send()
        for gx, gz in g_rdmas:
            gx.wait_send()
            gz.wait_send()
        for d in d_rdmas:
            d.wait_send()

    grid_spec = pltpu.PrefetchScalarGridSpec(
        num_scalar_prefetch=1,
        in_specs=[
            pl.BlockSpec(memory_space=pltpu.VMEM),
            pl.BlockSpec(memory_space=pl.ANY),
        ],
        out_specs=pl.BlockSpec(memory_space=pltpu.VMEM),
        scratch_shapes=[
            pltpu.VMEM((C, D), jnp.float32),
            pltpu.VMEM((C, D), jnp.bfloat16),
            pltpu.VMEM((C, D), jnp.bfloat16),
            pltpu.VMEM((C, D), jnp.bfloat16),
            pltpu.SemaphoreType.DMA((S,)),
            pltpu.SemaphoreType.DMA,
            pltpu.SemaphoreType.DMA((S,)),
            pltpu.SemaphoreType.DMA((S,)),
            pltpu.SemaphoreType.DMA((S,)),
            pltpu.SemaphoreType.DMA((S,)),
            pltpu.SemaphoreType.DMA((S,)),
            pltpu.SemaphoreType.DMA((S,)),
            pltpu.SemaphoreType.DMA((S,)),
            pltpu.SemaphoreType.DMA((S,)),
        ],
    )
    return pl.pallas_call(
        body,
        grid_spec=grid_spec,
        out_shape=jax.ShapeDtypeStruct((T, D), jnp.bfloat16),
        compiler_params=pltpu.CompilerParams(
            collective_id=0, vmem_limit_bytes=100 * 1024 * 1024
        ),
    )(ids, ids_col, E)


def kernel(ids, E):
    return _fused_kernel(ids, ids.reshape(T, 1), E)
